# baseline (speedup 1.0000x reference)
import jax
import jax.numpy as jnp
from jax.experimental import pallas as pl
from jax.experimental.pallas import tpu as pltpu

K_PAD = 1024
EMB = 64
B = 1024
C_BLK = 8
H = 32
W = 32
H_BLK = 2


def _scc_kernel(labels_ref, tableT_ref, out_ref, xT_ref):
    i = pl.program_id(0)
    j = pl.program_id(1)

    @pl.when((i == 0) & (j == 0))
    def _gather():
        labels = labels_ref[...]
        iota = jax.lax.broadcasted_iota(jnp.int32, (K_PAD, B), 0)
        onehotT = (iota == labels).astype(jnp.float32)
        xT_ref[...] = jnp.dot(
            tableT_ref[...], onehotT, preferred_element_type=jnp.float32
        )

    xs = xT_ref[pl.ds(i * C_BLK, C_BLK), :]
    out_ref[...] = jnp.broadcast_to(
        xs[:, None, None, :], (C_BLK, H_BLK, W, B)
    )


def kernel(class_labels, embedding_table):
    labels_row = class_labels.astype(jnp.int32).reshape(1, B)
    tableT = jnp.pad(
        embedding_table.T, ((0, 0), (0, K_PAD - embedding_table.shape[0]))
    )
    out = pl.pallas_call(
        _scc_kernel,
        grid=(EMB // C_BLK, H // H_BLK),
        in_specs=[
            pl.BlockSpec((1, B), lambda i, j: (0, 0)),
            pl.BlockSpec((EMB, K_PAD), lambda i, j: (0, 0)),
        ],
        out_specs=pl.BlockSpec((C_BLK, H_BLK, W, B), lambda i, j: (i, j, 0, 0)),
        out_shape=jax.ShapeDtypeStruct((EMB, H, W, B), jnp.float32),
        scratch_shapes=[pltpu.VMEM((EMB, B), jnp.float32)],
    )(labels_row, tableT)
    return jnp.transpose(out, (3, 0, 1, 2))


# speedup vs baseline: 1.0821x; 1.0821x over previous
import jax
import jax.numpy as jnp
from jax.experimental import pallas as pl
from jax.experimental.pallas import tpu as pltpu

K_PAD = 1024
EMB = 64
B = 1024
H = 32
W = 32


def _scc_kernel(labels_ref, tableT_ref, out_ref, xT_ref):
    i = pl.program_id(0)

    @pl.when(i == 0)
    def _gather():
        labels = labels_ref[...]
        iota = jax.lax.broadcasted_iota(jnp.int32, (K_PAD, B), 0)
        onehotT = (iota == labels).astype(jnp.float32)
        xT = jnp.dot(
            tableT_ref[...], onehotT, preferred_element_type=jnp.float32
        )
        xT_ref[...] = xT[:, None, :]

    val = xT_ref[i]
    out_ref[...] = jnp.broadcast_to(val[None, None, :, :], (1, H, W, B))


def kernel(class_labels, embedding_table):
    labels_row = class_labels.astype(jnp.int32).reshape(1, B)
    tableT = jnp.pad(
        embedding_table.T, ((0, 0), (0, K_PAD - embedding_table.shape[0]))
    )
    out = pl.pallas_call(
        _scc_kernel,
        grid=(EMB,),
        in_specs=[
            pl.BlockSpec((1, B), lambda i: (0, 0)),
            pl.BlockSpec((EMB, K_PAD), lambda i: (0, 0)),
        ],
        out_specs=pl.BlockSpec((1, H, W, B), lambda i: (i, 0, 0, 0)),
        out_shape=jax.ShapeDtypeStruct((EMB, H, W, B), jnp.float32),
        scratch_shapes=[pltpu.VMEM((EMB, 1, B), jnp.float32)],
    )(labels_row, tableT)
    return jnp.transpose(out, (3, 0, 1, 2))


# speedup vs baseline: 1.1535x; 1.0660x over previous
import jax
import jax.numpy as jnp
from jax.experimental import pallas as pl
from jax.experimental.pallas import tpu as pltpu

K_PAD = 1024
EMB = 64
B = 1024
C_BLK = 8
H = 32
W = 32
H_BLK = 4
NBUF = 4
N_J = H // H_BLK
N_STEPS = (EMB // C_BLK) * N_J


def _scc_kernel(labels_ref, tableT_ref, out_ref, xT_ref, buf_ref, sem):
    g = pl.program_id(0)
    i = g // N_J
    j = g % N_J
    s = g % NBUF

    @pl.when(g == 0)
    def _gather():
        labels = labels_ref[...]
        iota = jax.lax.broadcasted_iota(jnp.int32, (K_PAD, B), 0)
        onehotT = (iota == labels).astype(jnp.float32)
        xT_ref[...] = jnp.dot(
            tableT_ref[...], onehotT, preferred_element_type=jnp.float32
        )

    dst = out_ref.at[pl.ds(i * C_BLK, C_BLK), pl.ds(j * H_BLK, H_BLK)]
    copy = pltpu.make_async_copy(buf_ref.at[s], dst, sem.at[s])

    @pl.when(g >= NBUF)
    def _drain_slot():
        copy.wait()

    xs = xT_ref[pl.ds(i * C_BLK, C_BLK), :]
    buf_ref[s] = jnp.broadcast_to(xs[:, None, None, :], (C_BLK, H_BLK, W, B))
    copy.start()

    @pl.when(g == N_STEPS - 1)
    def _drain_all():
        for k in range(NBUF):
            pltpu.make_async_copy(buf_ref.at[k], dst, sem.at[k]).wait()


def kernel(class_labels, embedding_table):
    labels_row = class_labels.astype(jnp.int32).reshape(1, B)
    tableT = jnp.pad(
        embedding_table.T, ((0, 0), (0, K_PAD - embedding_table.shape[0]))
    )
    out = pl.pallas_call(
        _scc_kernel,
        grid=(N_STEPS,),
        in_specs=[
            pl.BlockSpec((1, B), lambda g: (0, 0)),
            pl.BlockSpec((EMB, K_PAD), lambda g: (0, 0)),
        ],
        out_specs=pl.BlockSpec(memory_space=pl.ANY),
        out_shape=jax.ShapeDtypeStruct((EMB, H, W, B), jnp.float32),
        scratch_shapes=[
            pltpu.VMEM((EMB, B), jnp.float32),
            pltpu.VMEM((NBUF, C_BLK, H_BLK, W, B), jnp.float32),
            pltpu.SemaphoreType.DMA((NBUF,)),
        ],
    )(labels_row, tableT)
    return jnp.transpose(out, (3, 0, 1, 2))


# speedup vs baseline: 1.1784x; 1.0216x over previous
import jax
import jax.numpy as jnp
from jax.experimental import pallas as pl
from jax.experimental.pallas import tpu as pltpu

K_PAD = 1024
EMB = 64
B = 1024
C_BLK = 16
H = 32
W = 32
H_BLK = 2


def _scc_kernel(labels_ref, tableT_ref, out_ref, xT_ref):
    i = pl.program_id(0)
    j = pl.program_id(1)

    @pl.when((i == 0) & (j == 0))
    def _gather():
        labels = labels_ref[...]
        iota = jax.lax.broadcasted_iota(jnp.int32, (K_PAD, B), 0)
        onehotT = (iota == labels).astype(jnp.float32)
        xT_ref[...] = jnp.dot(
            tableT_ref[...], onehotT, preferred_element_type=jnp.float32
        )

    xs = xT_ref[pl.ds(i * C_BLK, C_BLK), :]
    out_ref[...] = jnp.broadcast_to(
        xs[:, None, None, :], (C_BLK, H_BLK, W, B)
    )


def kernel(class_labels, embedding_table):
    labels_row = class_labels.astype(jnp.int32).reshape(1, B)
    tableT = jnp.pad(
        embedding_table.T, ((0, 0), (0, K_PAD - embedding_table.shape[0]))
    )
    out = pl.pallas_call(
        _scc_kernel,
        grid=(EMB // C_BLK, H // H_BLK),
        in_specs=[
            pl.BlockSpec((1, B), lambda i, j: (0, 0)),
            pl.BlockSpec((EMB, K_PAD), lambda i, j: (0, 0)),
        ],
        out_specs=pl.BlockSpec((C_BLK, H_BLK, W, B), lambda i, j: (i, j, 0, 0)),
        out_shape=jax.ShapeDtypeStruct((EMB, H, W, B), jnp.float32),
        scratch_shapes=[pltpu.VMEM((EMB, B), jnp.float32)],
    )(labels_row, tableT)
    return jnp.transpose(out, (3, 0, 1, 2))


# speedup vs baseline: 1.1888x; 1.0088x over previous
import jax
import jax.numpy as jnp
from jax.experimental import pallas as pl
from jax.experimental.pallas import tpu as pltpu

K_PAD = 1024
EMB = 64
B = 1024
C_BLK = 8
H = 32
W = 32
H_BLK = 4


def _scc_kernel(labels_ref, tableT_ref, out_ref, xT_ref):
    i = pl.program_id(0)
    j = pl.program_id(1)

    @pl.when((i == 0) & (j == 0))
    def _gather():
        labels = labels_ref[...]
        iota = jax.lax.broadcasted_iota(jnp.int32, (K_PAD, B), 0)
        onehotT = (iota == labels).astype(jnp.float32)
        xT_ref[...] = jnp.dot(
            tableT_ref[...], onehotT, preferred_element_type=jnp.float32
        )

    xs = xT_ref[pl.ds(i * C_BLK, C_BLK), :]
    out_ref[...] = jnp.broadcast_to(
        xs[:, None, None, :], (C_BLK, H_BLK, W, B)
    )


def kernel(class_labels, embedding_table):
    labels_row = class_labels.astype(jnp.int32).reshape(1, B)
    tableT = jnp.pad(
        embedding_table.T, ((0, 0), (0, K_PAD - embedding_table.shape[0]))
    )
    out = pl.pallas_call(
        _scc_kernel,
        grid=(EMB // C_BLK, H // H_BLK),
        in_specs=[
            pl.BlockSpec((1, B), lambda i, j: (0, 0)),
            pl.BlockSpec((EMB, K_PAD), lambda i, j: (0, 0)),
        ],
        out_specs=pl.BlockSpec((C_BLK, H_BLK, W, B), lambda i, j: (i, j, 0, 0)),
        out_shape=jax.ShapeDtypeStruct((EMB, H, W, B), jnp.float32),
        scratch_shapes=[pltpu.VMEM((EMB, B), jnp.float32)],
    )(labels_row, tableT)
    return jnp.transpose(out, (3, 0, 1, 2))
